# baseline (speedup 1.0000x reference)
import functools

import jax
import jax.numpy as jnp
from jax import lax
from jax.experimental import pallas as pl
from jax.experimental.pallas import tpu as pltpu
from jax.experimental.pallas import tpu_sc as plsc

BATCH = 16384
EMB_DIM = 4096
NUM_CLASSES = 1000
NUM_CORES = 2
NUM_SUBCORES = 16
NUM_TILES = NUM_CORES * NUM_SUBCORES
BPW = BATCH // NUM_TILES
NCACHE = 23
K = 4
CHW = K * EMB_DIM
UNROLL = 8
ADD_ITERS = EMB_DIM // 16 // UNROLL


def kernel(condition, image_emb, emb_table):
    cond = condition.astype(jnp.int32)
    tile_of = (cond * NUM_TILES) // NUM_CLASSES
    lo = (tile_of * NUM_CLASSES) // NUM_TILES
    hit = (cond - lo) < NCACHE
    key = tile_of * 2 + jnp.where(hit, 0, 1)
    pos = jnp.argsort(key, stable=True).astype(jnp.int32)
    cond_p = jnp.take(cond, pos, axis=0)

    img_flat = image_emb.reshape(-1)
    table_flat = emb_table.reshape(-1)
    mesh = plsc.VectorSubcoreMesh(core_axis_name="c", subcore_axis_name="s")

    @functools.partial(
        pl.kernel,
        mesh=mesh,
        out_type=jax.ShapeDtypeStruct((BATCH * EMB_DIM,), jnp.float32),
        scratch_types=[
            pltpu.VMEM((NCACHE * EMB_DIM,), jnp.float32),
            pltpu.VMEM((BPW,), jnp.int32),
            pltpu.VMEM((BPW,), jnp.int32),
            pltpu.VMEM((CHW,), jnp.float32),
            pltpu.VMEM((CHW,), jnp.float32),
            pltpu.SemaphoreType.DMA,
            pltpu.SemaphoreType.DMA,
            pltpu.SemaphoreType.DMA,
        ],
    )
    def run(cond_hbm, pos_hbm, img_hbm, table_hbm, out_hbm,
            cache_v, idx_v, pos_v, rows_v, img_v,
            sem_g, sem_i, sem_o):
        cid = lax.axis_index("c")
        sid = lax.axis_index("s")
        wid = cid * NUM_SUBCORES + sid
        lo_t = (wid * NUM_CLASSES) // NUM_TILES

        pltpu.sync_copy(
            table_hbm.at[pl.ds(lo_t * EMB_DIM, NCACHE * EMB_DIM)], cache_v
        )
        base = wid * BPW
        pltpu.sync_copy(cond_hbm.at[pl.ds(base, BPW)], idx_v)
        pltpu.sync_copy(pos_hbm.at[pl.ds(base, BPW)], pos_v)

        def drain(sem, nwords):
            pltpu.make_async_copy(
                img_hbm.at[pl.ds(0, nwords)],
                img_v.at[pl.ds(0, nwords)],
                sem,
            ).wait()

        def group_body(g, carry):
            iv = idx_v[pl.ds(g * 16, 16)]
            pv = pos_v[pl.ds(g * 16, 16)]
            for q in range(K):
                if q > 0:
                    drain(sem_o, CHW)
                else:
                    @pl.when(g > 0)
                    def _():
                        drain(sem_o, CHW)

                nm = jnp.int32(0)
                for u in range(K):
                    c = iv[q * K + u]
                    p = pv[q * K + u]
                    local = c - lo_t
                    is_hit = jnp.logical_and(local >= 0, local < NCACHE)
                    pltpu.async_copy(
                        img_hbm.at[pl.ds(p * EMB_DIM, EMB_DIM)],
                        img_v.at[pl.ds(u * EMB_DIM, EMB_DIM)],
                        sem_i,
                    )

                    def fetch_miss(c=c, u=u):
                        pltpu.async_copy(
                            table_hbm.at[pl.ds(c * EMB_DIM, EMB_DIM)],
                            rows_v.at[pl.ds(u * EMB_DIM, EMB_DIM)],
                            sem_g,
                        )

                    lax.cond(is_hit, lambda: None, fetch_miss)
                    nm = nm + lax.select(is_hit, jnp.int32(0), jnp.int32(1))

                drain(sem_i, CHW)
                for n in range(1, K + 1):
                    @pl.when(nm == n)
                    def _(n=n):
                        drain(sem_g, n * EMB_DIM)

                for u in range(K):
                    local = iv[q * K + u] - lo_t
                    is_hit = jnp.logical_and(local >= 0, local < NCACHE)

                    def add_from_cache(local=local, u=u):
                        cb = local * EMB_DIM

                        def ab(t, cc, u=u, cb=cb):
                            for uu in range(UNROLL):
                                off = (t * UNROLL + uu) * 16
                                sl = pl.ds(u * EMB_DIM + off, 16)
                                img_v[sl] = img_v[sl] + cache_v[pl.ds(cb + off, 16)]
                            return cc

                        lax.fori_loop(0, ADD_ITERS, ab, 0)

                    def add_from_rows(u=u):
                        def ab(t, cc, u=u):
                            for uu in range(UNROLL):
                                off = (t * UNROLL + uu) * 16
                                sl = pl.ds(u * EMB_DIM + off, 16)
                                img_v[sl] = img_v[sl] + rows_v[
                                    pl.ds(u * EMB_DIM + off, 16)
                                ]
                            return cc

                        lax.fori_loop(0, ADD_ITERS, ab, 0)

                    lax.cond(is_hit, add_from_cache, add_from_rows)

                for u in range(K):
                    p = pv[q * K + u]
                    pltpu.async_copy(
                        img_v.at[pl.ds(u * EMB_DIM, EMB_DIM)],
                        out_hbm.at[pl.ds(p * EMB_DIM, EMB_DIM)],
                        sem_o,
                    )
            return carry

        lax.fori_loop(0, BPW // 16, group_body, 0)
        drain(sem_o, CHW)

    out = run(cond_p, pos, img_flat, table_flat)
    return out.reshape(BATCH, EMB_DIM)


# speedup vs baseline: 1.8877x; 1.8877x over previous
import functools

import jax
import jax.numpy as jnp
from jax import lax
from jax.experimental import pallas as pl
from jax.experimental.pallas import tpu as pltpu
from jax.experimental.pallas import tpu_sc as plsc

BATCH = 16384
EMB_DIM = 4096
NUM_CLASSES = 1000
NUM_CORES = 2
NUM_SUBCORES = 16
NUM_TILES = NUM_CORES * NUM_SUBCORES
BPW = BATCH // NUM_TILES
NCACHE = 15
K = 8
CHW = K * EMB_DIM
UNROLL = 8
ADD_ITERS = EMB_DIM // 16 // UNROLL


def kernel(condition, image_emb, emb_table):
    cond = condition.astype(jnp.int32)
    tile_of = (cond * NUM_TILES) // NUM_CLASSES
    lo = (tile_of * NUM_CLASSES) // NUM_TILES
    hit = (cond - lo) < NCACHE
    key = tile_of * 2 + jnp.where(hit, 0, 1)
    pos = jnp.argsort(key, stable=True).astype(jnp.int32)
    cond_p = jnp.take(cond, pos, axis=0)

    table_flat = emb_table.reshape(-1)
    mesh = plsc.VectorSubcoreMesh(core_axis_name="c", subcore_axis_name="s")

    @functools.partial(
        pl.kernel,
        mesh=mesh,
        out_type=jax.ShapeDtypeStruct((BATCH, EMB_DIM), jnp.float32),
        scratch_types=[
            pltpu.VMEM((NCACHE * EMB_DIM,), jnp.float32),
            pltpu.VMEM((BPW,), jnp.int32),
            pltpu.VMEM((BPW,), jnp.int32),
            pltpu.VMEM((K, EMB_DIM), jnp.float32),
            pltpu.VMEM((K, EMB_DIM), jnp.float32),
            pltpu.SemaphoreType.DMA,
            pltpu.SemaphoreType.DMA,
            pltpu.SemaphoreType.DMA,
        ],
    )
    def run(cond_hbm, pos_hbm, img_hbm, table_hbm, tflat_hbm, out_hbm,
            cache_v, idx_v, pos_v, rows_v, img_v,
            sem_g, sem_i, sem_o):
        cid = lax.axis_index("c")
        sid = lax.axis_index("s")
        wid = cid * NUM_SUBCORES + sid
        lo_t = (wid * NUM_CLASSES) // NUM_TILES

        pltpu.sync_copy(
            tflat_hbm.at[pl.ds(lo_t * EMB_DIM, NCACHE * EMB_DIM)], cache_v
        )
        base = wid * BPW
        pltpu.sync_copy(cond_hbm.at[pl.ds(base, BPW)], idx_v)
        pltpu.sync_copy(pos_hbm.at[pl.ds(base, BPW)], pos_v)

        def drain(sem, buf):
            pltpu.make_async_copy(
                table_hbm.at[pl.ds(0, K)], buf, sem
            ).wait()

        def group_body(g, carry):
            iv = idx_v[pl.ds(g * 16, 16)]
            for half in range(2):
                j = g * 2 + half
                if half == 1:
                    drain(sem_o, img_v)
                else:
                    @pl.when(g > 0)
                    def _():
                        drain(sem_o, img_v)

                pltpu.async_copy(
                    img_hbm.at[pos_v.at[pl.ds(j * K, K)]], img_v, sem_i
                )

                l_first = iv[half * K] - lo_t
                l_last = iv[half * K + 7] - lo_t
                all_hit = jnp.logical_and(
                    jnp.logical_and(l_first >= 0, l_first < NCACHE),
                    jnp.logical_and(l_last >= 0, l_last < NCACHE),
                )

                locals_u = [iv[half * K + u] - lo_t for u in range(K)]

                def hit_mid(locals_u=locals_u):
                    pltpu.make_async_copy(
                        table_hbm.at[pl.ds(0, K)], img_v, sem_i
                    ).wait()
                    for u in range(K):
                        cb = locals_u[u] * EMB_DIM

                        def ab(t, cc, u=u, cb=cb):
                            for uu in range(UNROLL):
                                off = (t * UNROLL + uu) * 16
                                img_v[u, pl.ds(off, 16)] = (
                                    img_v[u, pl.ds(off, 16)]
                                    + cache_v[pl.ds(cb + off, 16)]
                                )
                            return cc

                        lax.fori_loop(0, ADD_ITERS, ab, 0)

                def miss_mid(j=j):
                    pltpu.async_copy(
                        table_hbm.at[idx_v.at[pl.ds(j * K, K)]], rows_v, sem_g
                    )
                    drain(sem_g, rows_v)
                    pltpu.make_async_copy(
                        table_hbm.at[pl.ds(0, K)], img_v, sem_i
                    ).wait()
                    for u in range(K):
                        def ab(t, cc, u=u):
                            for uu in range(UNROLL):
                                off = (t * UNROLL + uu) * 16
                                img_v[u, pl.ds(off, 16)] = (
                                    img_v[u, pl.ds(off, 16)]
                                    + rows_v[u, pl.ds(off, 16)]
                                )
                            return cc

                        lax.fori_loop(0, ADD_ITERS, ab, 0)

                lax.cond(all_hit, hit_mid, miss_mid)

                pltpu.async_copy(
                    img_v, out_hbm.at[pos_v.at[pl.ds(j * K, K)]], sem_o
                )
            return carry

        lax.fori_loop(0, BPW // 16, group_body, 0)
        drain(sem_o, img_v)

    return run(cond_p, pos, image_emb, emb_table, table_flat)


# speedup vs baseline: 2.6522x; 1.4050x over previous
import functools

import jax
import jax.numpy as jnp
from jax import lax
from jax.experimental import pallas as pl
from jax.experimental.pallas import tpu as pltpu
from jax.experimental.pallas import tpu_sc as plsc

BATCH = 16384
EMB_DIM = 4096
NUM_CLASSES = 1000
NUM_CORES = 2
NUM_SUBCORES = 16
NUM_TILES = NUM_CORES * NUM_SUBCORES
BPW = BATCH // NUM_TILES
NCACHE = 15
K = 8
CHW = K * EMB_DIM
UNROLL = 8
ADD_ITERS = EMB_DIM // 16 // UNROLL


def kernel(condition, image_emb, emb_table):
    cond = condition.astype(jnp.int32)
    tile_of = (cond * NUM_TILES) // NUM_CLASSES
    lo = (tile_of * NUM_CLASSES) // NUM_TILES
    hit = (cond - lo) < NCACHE
    key = tile_of * 2 + jnp.where(hit, 0, 1)
    pos = jnp.argsort(key, stable=True).astype(jnp.int32)
    cond_p = jnp.take(cond, pos, axis=0)

    table_flat = emb_table.reshape(-1)
    mesh = plsc.VectorSubcoreMesh(core_axis_name="c", subcore_axis_name="s")

    @functools.partial(
        pl.kernel,
        mesh=mesh,
        out_type=jax.ShapeDtypeStruct((BATCH, EMB_DIM), jnp.float32),
        scratch_types=[
            pltpu.VMEM((NCACHE * EMB_DIM,), jnp.float32),
            pltpu.VMEM((BPW,), jnp.int32),
            pltpu.VMEM((BPW,), jnp.int32),
            pltpu.VMEM((K, EMB_DIM), jnp.float32),
            pltpu.VMEM((K, EMB_DIM), jnp.float32),
            pltpu.SemaphoreType.DMA,
            pltpu.SemaphoreType.DMA,
            pltpu.SemaphoreType.DMA,
        ],
    )
    def run(cond_hbm, pos_hbm, img_hbm, table_hbm, tflat_hbm, out_hbm,
            cache_v, idx_v, pos_v, rows_v, img_v,
            sem_g, sem_i, sem_o):
        cid = lax.axis_index("c")
        sid = lax.axis_index("s")
        wid = cid * NUM_SUBCORES + sid
        lo_t = (wid * NUM_CLASSES) // NUM_TILES

        pltpu.sync_copy(
            tflat_hbm.at[pl.ds(lo_t * EMB_DIM, NCACHE * EMB_DIM)], cache_v
        )
        base = wid * BPW
        pltpu.sync_copy(cond_hbm.at[pl.ds(base, BPW)], idx_v)
        pltpu.sync_copy(pos_hbm.at[pl.ds(base, BPW)], pos_v)

        def drain(sem, buf):
            pltpu.make_async_copy(
                table_hbm.at[pl.ds(0, K)], buf, sem
            ).wait()

        def group_body(g, carry):
            iv = idx_v[pl.ds(g * 16, 16)]
            for half in range(2):
                j = g * 2 + half
                if half == 1:
                    drain(sem_o, img_v)
                else:
                    @pl.when(g > 0)
                    def _():
                        drain(sem_o, img_v)

                pltpu.async_copy(
                    img_hbm.at[pos_v.at[pl.ds(j * K, K)]], img_v, sem_i
                )

                l_first = iv[half * K] - lo_t
                l_last = iv[half * K + 7] - lo_t
                all_hit = jnp.logical_and(
                    jnp.logical_and(l_first >= 0, l_first < NCACHE),
                    jnp.logical_and(l_last >= 0, l_last < NCACHE),
                )

                locals_u = [iv[half * K + u] - lo_t for u in range(K)]

                def hit_mid(locals_u=locals_u):
                    pltpu.make_async_copy(
                        table_hbm.at[pl.ds(0, K)], img_v, sem_i
                    ).wait()
                    for u in range(K):
                        cb = locals_u[u] * EMB_DIM

                        def ab(t, cc, u=u, cb=cb):
                            for uu in range(UNROLL):
                                off = (t * UNROLL + uu) * 16
                                img_v[u, pl.ds(off, 16)] = (
                                    img_v[u, pl.ds(off, 16)]
                                    + cache_v[pl.ds(cb + off, 16)]
                                )
                            return cc

                        lax.fori_loop(0, ADD_ITERS, ab, 0)

                def miss_mid(j=j):
                    pltpu.async_copy(
                        table_hbm.at[idx_v.at[pl.ds(j * K, K)]], rows_v, sem_g
                    )
                    drain(sem_g, rows_v)
                    pltpu.make_async_copy(
                        table_hbm.at[pl.ds(0, K)], img_v, sem_i
                    ).wait()
                    for u in range(K):
                        def ab(t, cc, u=u):
                            for uu in range(UNROLL):
                                off = (t * UNROLL + uu) * 16
                                img_v[u, pl.ds(off, 16)] = (
                                    img_v[u, pl.ds(off, 16)]
                                    + rows_v[u, pl.ds(off, 16)]
                                )
                            return cc

                        lax.fori_loop(0, ADD_ITERS, ab, 0)

                del all_hit, hit_mid
                miss_mid()

                pltpu.async_copy(
                    img_v, out_hbm.at[pos_v.at[pl.ds(j * K, K)]], sem_o
                )
            return carry

        lax.fori_loop(0, BPW // 16, group_body, 0)
        drain(sem_o, img_v)

    return run(cond_p, pos, image_emb, emb_table, table_flat)
